# baseline (speedup 1.0000x reference)
import jax
import jax.numpy as jnp
from jax.experimental import pallas as pl
from jax.experimental.pallas import tpu as pltpu

B, N, DIN, DOUT, K = 32, 256, 256, 128, 8
CPK = DOUT // K
KP = K // 2


def _body(x_ref, adj_ref, w_ref, bias_ref, out_ref, h_ref):
    j = pl.program_id(1)

    @pl.when(j == 0)
    def _compute_h():
        hh = jnp.dot(x_ref[0], w_ref[...], preferred_element_type=jnp.float32)
        h_ref[...] = jnp.maximum(hh + bias_ref[...], 0.0)

    h = h_ref[...]
    a = adj_ref[0].reshape(KP * N, N)
    r = jnp.dot(a, h, preferred_element_type=jnp.float32)
    rr = r.reshape(KP, N, DOUT)
    g_local = (jax.lax.broadcasted_iota(jnp.int32, (N, DOUT), 1) // CPK
               - j * KP)
    acc = rr[0]
    for i in range(1, KP):
        acc = jnp.where(g_local == i, rr[i], acc)

    @pl.when(j == 0)
    def _write_first():
        out_ref[0] = acc

    @pl.when(j == 1)
    def _write_second():
        out_ref[0] = jnp.where(g_local < 0, out_ref[0], acc)


def kernel(node_feats, adj, W, b):
    bias = b.reshape(1, DOUT)
    out = pl.pallas_call(
        _body,
        grid=(B, 2),
        in_specs=[
            pl.BlockSpec((1, N, DIN), lambda i, j: (i, 0, 0)),
            pl.BlockSpec((1, KP, N, N), lambda i, j: (i, j, 0, 0)),
            pl.BlockSpec((DIN, DOUT), lambda i, j: (0, 0)),
            pl.BlockSpec((1, DOUT), lambda i, j: (0, 0)),
        ],
        out_specs=pl.BlockSpec((1, N, DOUT), lambda i, j: (i, 0, 0)),
        out_shape=jax.ShapeDtypeStruct((B, N, DOUT), jnp.float32),
        scratch_shapes=[pltpu.VMEM((N, DOUT), jnp.float32)],
        compiler_params=pltpu.CompilerParams(
            dimension_semantics=("arbitrary", "arbitrary"),
        ),
    )(node_feats, adj, W, bias)
    return out


# speedup vs baseline: 1.6237x; 1.6237x over previous
import jax
import jax.numpy as jnp
from jax.experimental import pallas as pl
from jax.experimental.pallas import tpu as pltpu

B, N, DIN, DOUT, K = 32, 256, 256, 128, 8
CPK = DOUT // K
NBUF = 3


def _body(x_ref, adj_hbm, w_ref, bias_ref, out_ref, bufs, sems):
    b = pl.program_id(0)

    @pl.when(b == 0)
    def _prologue():
        for d in range(NBUF):
            pltpu.make_async_copy(adj_hbm.at[d], bufs.at[d], sems.at[d]).start()

    slot = jax.lax.rem(b, NBUF)
    pltpu.make_async_copy(adj_hbm.at[b], bufs.at[slot], sems.at[slot]).wait()

    h = jnp.dot(x_ref[0], w_ref[...], preferred_element_type=jnp.float32)
    h = jnp.maximum(h + bias_ref[...], 0.0)
    a = bufs[slot].reshape(K * N, N)
    r = jnp.dot(a, h, preferred_element_type=jnp.float32)
    rr = r.reshape(K, N, DOUT)
    lane_group = jax.lax.broadcasted_iota(jnp.int32, (N, DOUT), 1) // CPK
    acc = rr[0]
    for k in range(1, K):
        acc = jnp.where(lane_group == k, rr[k], acc)
    out_ref[0] = acc

    @pl.when(b + NBUF < B)
    def _prefetch():
        nb = b + NBUF
        nslot = jax.lax.rem(nb, NBUF)
        pltpu.make_async_copy(adj_hbm.at[nb], bufs.at[nslot],
                              sems.at[nslot]).start()


def kernel(node_feats, adj, W, b):
    bias = b.reshape(1, DOUT)
    out = pl.pallas_call(
        _body,
        grid=(B,),
        in_specs=[
            pl.BlockSpec((1, N, DIN), lambda i: (i, 0, 0)),
            pl.BlockSpec(memory_space=pltpu.MemorySpace.HBM),
            pl.BlockSpec((DIN, DOUT), lambda i: (0, 0)),
            pl.BlockSpec((1, DOUT), lambda i: (0, 0)),
        ],
        out_specs=pl.BlockSpec((1, N, DOUT), lambda i: (i, 0, 0)),
        out_shape=jax.ShapeDtypeStruct((B, N, DOUT), jnp.float32),
        scratch_shapes=[
            pltpu.VMEM((NBUF, K, N, N), jnp.float32),
            pltpu.SemaphoreType.DMA((NBUF,)),
        ],
        compiler_params=pltpu.CompilerParams(
            dimension_semantics=("arbitrary",),
        ),
    )(node_feats, adj, W, bias)
    return out


# speedup vs baseline: 1.6895x; 1.0405x over previous
import jax
import jax.numpy as jnp
from jax.experimental import pallas as pl
from jax.experimental.pallas import tpu as pltpu

B, N, DIN, DOUT, K = 32, 256, 256, 128, 8
CPK = DOUT // K
NBUF = 4
KH = K // 2


def _issue(adj_hbm, bufs, sems, bb):
    nslot = jax.lax.rem(bb, NBUF)
    pltpu.make_async_copy(adj_hbm.at[bb, pl.ds(0, KH)],
                          bufs.at[nslot, pl.ds(0, KH)],
                          sems.at[nslot, 0]).start()
    pltpu.make_async_copy(adj_hbm.at[bb, pl.ds(KH, KH)],
                          bufs.at[nslot, pl.ds(KH, KH)],
                          sems.at[nslot, 1]).start()


def _body(x_ref, adj_hbm, w_ref, bias_ref, out_ref, bufs, sems):
    b = pl.program_id(0)

    @pl.when(b == 0)
    def _prologue():
        for d in range(NBUF - 1):
            _issue(adj_hbm, bufs, sems, d)

    @pl.when(b + NBUF - 1 < B)
    def _prefetch():
        _issue(adj_hbm, bufs, sems, b + NBUF - 1)

    slot = jax.lax.rem(b, NBUF)
    h = jnp.dot(x_ref[0], w_ref[...], preferred_element_type=jnp.float32)
    h = jnp.maximum(h + bias_ref[...], 0.0)

    lane_group = jax.lax.broadcasted_iota(jnp.int32, (N, DOUT), 1) // CPK

    pltpu.make_async_copy(adj_hbm.at[b, pl.ds(0, KH)],
                          bufs.at[slot, pl.ds(0, KH)],
                          sems.at[slot, 0]).wait()
    r0 = jnp.dot(bufs[slot, :KH].reshape(KH * N, N), h,
                 preferred_element_type=jnp.float32).reshape(KH, N, DOUT)

    pltpu.make_async_copy(adj_hbm.at[b, pl.ds(KH, KH)],
                          bufs.at[slot, pl.ds(KH, KH)],
                          sems.at[slot, 1]).wait()
    r1 = jnp.dot(bufs[slot, KH:].reshape(KH * N, N), h,
                 preferred_element_type=jnp.float32).reshape(KH, N, DOUT)

    acc = r0[0]
    for k in range(1, KH):
        acc = jnp.where(lane_group == k, r0[k], acc)
    for k in range(KH):
        acc = jnp.where(lane_group == KH + k, r1[k], acc)
    out_ref[0] = acc


def kernel(node_feats, adj, W, b):
    bias = b.reshape(1, DOUT)
    out = pl.pallas_call(
        _body,
        grid=(B,),
        in_specs=[
            pl.BlockSpec((1, N, DIN), lambda i: (i, 0, 0)),
            pl.BlockSpec(memory_space=pltpu.MemorySpace.HBM),
            pl.BlockSpec((DIN, DOUT), lambda i: (0, 0)),
            pl.BlockSpec((1, DOUT), lambda i: (0, 0)),
        ],
        out_specs=pl.BlockSpec((1, N, DOUT), lambda i: (i, 0, 0)),
        out_shape=jax.ShapeDtypeStruct((B, N, DOUT), jnp.float32),
        scratch_shapes=[
            pltpu.VMEM((NBUF, K, N, N), jnp.float32),
            pltpu.SemaphoreType.DMA((NBUF, 2)),
        ],
        compiler_params=pltpu.CompilerParams(
            dimension_semantics=("arbitrary",),
        ),
    )(node_feats, adj, W, bias)
    return out
